# baseline (speedup 1.0000x reference)
import functools

import jax
import jax.numpy as jnp
from jax import lax
from jax.experimental import pallas as pl
from jax.experimental.pallas import tpu as pltpu
from jax.experimental.pallas import tpu_sc as plsc

NUM_NODES = 1000000
D = 64
B = 16384
NC = 2
NS = 16
L = 16
NW = NC * NS
B_PER_W = B // NW
NCHUNK = D // L


def _body(mem_hbm, ids_hbm, t_hbm, w_hbm, bias_hbm, out_hbm,
          idx_v, t_v, rows_v, w_v, bias_v, sem):
    wid = lax.axis_index("s") * NC + lax.axis_index("c")
    base = wid * B_PER_W

    pltpu.sync_copy(ids_hbm.at[pl.ds(base, B_PER_W)], idx_v)
    pltpu.sync_copy(t_hbm.at[pl.ds(base, B_PER_W)], t_v)
    pltpu.sync_copy(w_hbm, w_v)
    pltpu.sync_copy(bias_hbm, bias_v)

    pltpu.async_copy(mem_hbm.at[idx_v], rows_v, sem).wait()

    w_c = [w_v[pl.ds(c * L, L)] for c in range(NCHUNK)]
    b1_c = [bias_v[pl.ds(c * L, L)] + 1.0 for c in range(NCHUNK)]

    def blk_fn(blk, carry):
        row0 = blk * L
        t_blk = t_v[pl.ds(row0, L)]
        for j in range(L):
            i = row0 + j
            t_i = t_blk[j]
            for c in range(NCHUNK):
                sl = pl.ds(c * L, L)
                rows_v[i, sl] = rows_v[i, sl] * (t_i * w_c[c] + b1_c[c])
        return carry

    lax.fori_loop(0, B_PER_W // L, blk_fn, 0)

    pltpu.sync_copy(rows_v, out_hbm.at[pl.ds(base, B_PER_W)])


@jax.jit
def _tpe(node_memories, node_ids, node_time_intervals, W, b):
    mesh = plsc.VectorSubcoreMesh(core_axis_name="c", subcore_axis_name="s")
    return pl.kernel(
        _body,
        out_type=jax.ShapeDtypeStruct((B, D), jnp.float32),
        mesh=mesh,
        scratch_types=[
            pltpu.VMEM((B_PER_W,), jnp.int32),
            pltpu.VMEM((B_PER_W,), jnp.float32),
            pltpu.VMEM((B_PER_W, D), jnp.float32),
            pltpu.VMEM((D,), jnp.float32),
            pltpu.VMEM((D,), jnp.float32),
            pltpu.SemaphoreType.DMA,
        ],
        compiler_params=pltpu.CompilerParams(use_tc_tiling_on_sc=False),
    )(node_memories, node_ids, node_time_intervals, W, b)


def kernel(node_memories, node_ids, node_time_intervals, W, b):
    return _tpe(node_memories, node_ids.astype(jnp.int32),
                node_time_intervals, W, b)


# speedup vs baseline: 1.1115x; 1.1115x over previous
import jax
import jax.numpy as jnp
from jax import lax
from jax.experimental import pallas as pl
from jax.experimental.pallas import tpu as pltpu
from jax.experimental.pallas import tpu_sc as plsc

B = 16384
D = 64
L = 16
NW = 32
BPW = B // NW
H = BPW // 2


def _body(tab2, ids, t, w, bias, out, idx_v, t_v, rows_v, o_v, w_v, b1_v, sem):
    wid = lax.axis_index("s") * 2 + lax.axis_index("c")
    base = wid * BPW
    pltpu.sync_copy(ids.at[pl.ds(base, BPW)], idx_v)
    pltpu.sync_copy(t.at[pl.ds(base, BPW)], t_v)
    pltpu.sync_copy(w, w_v)
    pltpu.sync_copy(bias, b1_v)

    w_c = [w_v[pl.ds(c * L, L)] for c in range(4)]
    b1_c = [b1_v[pl.ds(c * L, L)] + 1.0 for c in range(4)]

    for h in range(2):
        hb = h * H
        pltpu.async_copy(tab2.at[idx_v.at[pl.ds(hb, H)]], rows_v, sem).wait()

        def blk(bi, carry):
            r0 = bi * L
            t_blk = t_v[pl.ds(hb + r0, L)]
            for j in range(L):
                i = r0 + j
                t_i = t_blk[j]
                for c in range(4):
                    g = rows_v[i, pl.ds(c * L, L)]
                    o_v[i, pl.ds(c * L, L)] = g * (t_i * w_c[c] + b1_c[c])
            return carry
        lax.fori_loop(0, H // L, blk, 0)

        pltpu.sync_copy(o_v, out.at[pl.ds(base + hb, H)])


@jax.jit
def _tpe(tab2, ids, t, w, bias):
    mesh = plsc.VectorSubcoreMesh(core_axis_name="c", subcore_axis_name="s")
    return pl.kernel(
        _body,
        out_type=jax.ShapeDtypeStruct((B, D), jnp.float32),
        mesh=mesh,
        scratch_types=[
            pltpu.VMEM((BPW,), jnp.int32),
            pltpu.VMEM((BPW,), jnp.float32),
            pltpu.VMEM((H, 128), jnp.float32),
            pltpu.VMEM((H, D), jnp.float32),
            pltpu.VMEM((D,), jnp.float32),
            pltpu.VMEM((D,), jnp.float32),
            pltpu.SemaphoreType.DMA,
        ],
    )(tab2, ids, t, w, bias)


def kernel(node_memories, node_ids, node_time_intervals, W, b):
    tab2 = jnp.pad(node_memories, ((0, 0), (0, 64)))
    return _tpe(tab2, node_ids.astype(jnp.int32), node_time_intervals, W, b)


# speedup vs baseline: 1.5378x; 1.3836x over previous
import jax
import jax.numpy as jnp
from jax import lax
from jax.experimental import pallas as pl
from jax.experimental.pallas import tpu as pltpu
from jax.experimental.pallas import tpu_sc as plsc

B = 16384
D = 64
L = 16
NW = 32
BPW = B // NW
NBUF = 8
OCH = 128


def _body(tab, ids, t, w, bias, outv, idx_v, t_v, o_row, w_v, b1_v, bufs, sems):
    wid = lax.axis_index("s") * 2 + lax.axis_index("c")
    base = wid * BPW
    pltpu.sync_copy(ids.at[pl.ds(base, BPW)], idx_v)
    pltpu.sync_copy(t.at[pl.ds(base, BPW)], t_v)
    pltpu.sync_copy(w, w_v)
    pltpu.sync_copy(bias, b1_v)

    w_c = [w_v[pl.ds(c * L, L)] for c in range(4)]
    b1_c = [b1_v[pl.ds(c * L, L)] + 1.0 for c in range(4)]

    def issue(chunk, off, slot):
        v = idx_v[pl.ds(chunk * L, L)]
        i = v[off]
        row0 = pl.multiple_of((i >> 3) << 3, 8)
        pltpu.async_copy(tab.at[pl.ds(row0, 8), :], bufs[slot], sems[slot])

    for j0 in range(NBUF):
        issue(0, j0, j0)

    def blk(n, carry):
        v = idx_v[pl.ds(n * L, L)]
        t_blk = t_v[pl.ds(n * L, L)]
        for j in range(L):
            slot = j % NBUF
            pltpu.make_async_copy(tab.at[pl.ds(0, 8), :], bufs[slot],
                                  sems[slot]).wait()
            r = v[j] & 7
            t_k = t_blk[j]
            orow = (n % 8) * L + j
            for c in range(4):
                g = bufs[slot][r, pl.ds(c * L, L)]
                o_row[orow, pl.ds(c * L, L)] = g * (t_k * w_c[c] + b1_c[c])
            nxt = n * L + j + NBUF
            @pl.when(nxt < BPW)
            def _():
                issue((n * L + j + NBUF) // L, (j + NBUF) % L, slot)
        @pl.when(n % 8 == 7)
        def _():
            pltpu.sync_copy(o_row, outv.at[pl.ds(base + (n // 8) * OCH, OCH)])
        return carry

    lax.fori_loop(0, BPW // L, blk, 0)


@jax.jit
def _tpe(tab, ids, t, w, bias):
    mesh = plsc.VectorSubcoreMesh(core_axis_name="c", subcore_axis_name="s")
    return pl.kernel(
        _body,
        out_type=jax.ShapeDtypeStruct((B, D), jnp.float32),
        mesh=mesh,
        scratch_types=[
            pltpu.VMEM((BPW,), jnp.int32),
            pltpu.VMEM((BPW,), jnp.float32),
            pltpu.VMEM((OCH, D), jnp.float32),
            pltpu.VMEM((D,), jnp.float32),
            pltpu.VMEM((D,), jnp.float32),
            [pltpu.VMEM((8, D), jnp.float32) for _ in range(NBUF)],
            [pltpu.SemaphoreType.DMA for _ in range(NBUF)],
        ],
    )(tab, ids, t, w, bias)


def kernel(node_memories, node_ids, node_time_intervals, W, b):
    return _tpe(node_memories, node_ids.astype(jnp.int32),
                node_time_intervals, W, b)


# speedup vs baseline: 2.5239x; 1.6413x over previous
import jax
import jax.numpy as jnp
from jax import lax
from jax.experimental import pallas as pl
from jax.experimental.pallas import tpu as pltpu
from jax.experimental.pallas import tpu_sc as plsc

B = 16384
D = 64
L = 16
NW = 32
BPW = B // NW
NBUF = 4
OCH = 128


def _body(tabT, ids, t, w, bias, outv, idx_v, t_v, o_row, w_v, b1_v, bufs, sems):
    wid = lax.axis_index("s") * 2 + lax.axis_index("c")
    base = wid * BPW
    pltpu.sync_copy(ids.at[pl.ds(base, BPW)], idx_v)
    pltpu.sync_copy(t.at[pl.ds(base, BPW)], t_v)
    pltpu.sync_copy(w, w_v)
    pltpu.sync_copy(bias, b1_v)

    w_c = [w_v[pl.ds(c * L, L)] for c in range(4)]
    b1_c = [b1_v[pl.ds(c * L, L)] + 1.0 for c in range(4)]
    iota = lax.iota(jnp.int32, L)
    rows_c = [c * L + iota for c in range(4)]

    def issue(chunk, off, slot):
        v = idx_v[pl.ds(chunk * L, L)]
        i = v[off]
        col0 = pl.multiple_of((i >> 7) << 7, 128)
        pltpu.async_copy(tabT.at[:, pl.ds(col0, 128)], bufs[slot], sems[slot])

    for j0 in range(NBUF):
        issue(0, j0, j0)

    def blk(n, carry):
        v = idx_v[pl.ds(n * L, L)]
        t_blk = t_v[pl.ds(n * L, L)]
        for j in range(L):
            slot = j % NBUF
            pltpu.make_async_copy(tabT.at[:, pl.ds(0, 128)], bufs[slot],
                                  sems[slot]).wait()
            lane = v[j] & 127
            cols = jnp.broadcast_to(lane, (L,))
            t_k = t_blk[j]
            orow = (n % 8) * L + j
            for c in range(4):
                g = plsc.load_gather(bufs[slot], [rows_c[c], cols])
                o_row[orow, pl.ds(c * L, L)] = g * (t_k * w_c[c] + b1_c[c])
            nxt = n * L + j + NBUF
            @pl.when(nxt < BPW)
            def _():
                issue((n * L + j + NBUF) // L, (j + NBUF) % L, slot)
        @pl.when(n % 8 == 7)
        def _():
            pltpu.sync_copy(o_row, outv.at[pl.ds(base + (n // 8) * OCH, OCH)])
        return carry

    lax.fori_loop(0, BPW // L, blk, 0)


@jax.jit
def _tpe(tabT, ids, t, w, bias):
    mesh = plsc.VectorSubcoreMesh(core_axis_name="c", subcore_axis_name="s")
    return pl.kernel(
        _body,
        out_type=jax.ShapeDtypeStruct((B, D), jnp.float32),
        mesh=mesh,
        scratch_types=[
            pltpu.VMEM((BPW,), jnp.int32),
            pltpu.VMEM((BPW,), jnp.float32),
            pltpu.VMEM((OCH, D), jnp.float32),
            pltpu.VMEM((D,), jnp.float32),
            pltpu.VMEM((D,), jnp.float32),
            [pltpu.VMEM((D, 128), jnp.float32) for _ in range(NBUF)],
            [pltpu.SemaphoreType.DMA for _ in range(NBUF)],
        ],
        compiler_params=pltpu.CompilerParams(needs_layout_passes=False),
    )(tabT, ids, t, w, bias)


def kernel(node_memories, node_ids, node_time_intervals, W, b):
    tabT = jnp.swapaxes(node_memories, 0, 1)
    return _tpe(tabT, node_ids.astype(jnp.int32), node_time_intervals, W, b)


# speedup vs baseline: 2.9528x; 1.1699x over previous
import jax
import jax.numpy as jnp
from jax import lax
from jax.experimental import pallas as pl
from jax.experimental.pallas import tpu as pltpu
from jax.experimental.pallas import tpu_sc as plsc

B = 16384
D = 64
L = 16
NW = 32
BPW = B // NW
NBUF = 8
OCH = 128


def _body(tabT, ids, t, w, bias, outv, idx_v, t_v, o_row, w_v, b1_v, bufs, sems):
    wid = lax.axis_index("s") * 2 + lax.axis_index("c")
    base = wid * BPW
    pltpu.sync_copy(ids.at[pl.ds(base, BPW)], idx_v)
    pltpu.sync_copy(t.at[pl.ds(base, BPW)], t_v)
    pltpu.sync_copy(w, w_v)
    pltpu.sync_copy(bias, b1_v)

    w_c = [w_v[pl.ds(c * L, L)] for c in range(4)]
    b1_c = [b1_v[pl.ds(c * L, L)] + 1.0 for c in range(4)]
    iota = lax.iota(jnp.int32, L)
    rows_c = [c * L + iota for c in range(4)]

    def issue(chunk, off, slot):
        v = idx_v[pl.ds(chunk * L, L)]
        i = v[off]
        col0 = pl.multiple_of((i >> 7) << 7, 128)
        pltpu.async_copy(tabT.at[:, pl.ds(col0, 128)], bufs[slot], sems[slot])

    for j0 in range(NBUF):
        issue(0, j0, j0)

    def blk(n, carry):
        v = idx_v[pl.ds(n * L, L)]
        t_blk = t_v[pl.ds(n * L, L)]
        for j in range(L):
            slot = j % NBUF
            pltpu.make_async_copy(tabT.at[:, pl.ds(0, 128)], bufs[slot],
                                  sems[slot]).wait()
            lane = v[j] & 127
            cols = jnp.broadcast_to(lane, (L,))
            t_k = t_blk[j]
            orow = (n % 8) * L + j
            for c in range(4):
                g = plsc.load_gather(bufs[slot], [rows_c[c], cols])
                o_row[orow, pl.ds(c * L, L)] = g * (t_k * w_c[c] + b1_c[c])
            nxt = n * L + j + NBUF
            @pl.when(nxt < BPW)
            def _():
                issue((n * L + j + NBUF) // L, (j + NBUF) % L, slot)
        @pl.when(n % 8 == 7)
        def _():
            pltpu.sync_copy(o_row, outv.at[pl.ds(base + (n // 8) * OCH, OCH)])
        return carry

    lax.fori_loop(0, BPW // L, blk, 0)


@jax.jit
def _tpe(tabT, ids, t, w, bias):
    mesh = plsc.VectorSubcoreMesh(core_axis_name="c", subcore_axis_name="s")
    return pl.kernel(
        _body,
        out_type=jax.ShapeDtypeStruct((B, D), jnp.float32),
        mesh=mesh,
        scratch_types=[
            pltpu.VMEM((BPW,), jnp.int32),
            pltpu.VMEM((BPW,), jnp.float32),
            pltpu.VMEM((OCH, D), jnp.float32),
            pltpu.VMEM((D,), jnp.float32),
            pltpu.VMEM((D,), jnp.float32),
            [pltpu.VMEM((D, 128), jnp.float32) for _ in range(NBUF)],
            [pltpu.SemaphoreType.DMA for _ in range(NBUF)],
        ],
        compiler_params=pltpu.CompilerParams(needs_layout_passes=False),
    )(tabT, ids, t, w, bias)


def kernel(node_memories, node_ids, node_time_intervals, W, b):
    tabT = jnp.swapaxes(node_memories, 0, 1)
    return _tpe(tabT, node_ids.astype(jnp.int32), node_time_intervals, W, b)
